# baseline (speedup 1.0000x reference)
import functools
import math

import jax
import jax.numpy as jnp
import numpy as np
from jax import lax
from jax.experimental import pallas as pl
from jax.experimental.pallas import tpu as pltpu
from jax.experimental.pallas import tpu_sc as plsc

_NUM_NEG = 29
_NCAND = _NUM_NEG + 1
_KS = (1, 5, 10)
_LANES = 16
_NUM_CORES = 2
_NUM_SUBCORES = 16
_NW = _NUM_CORES * _NUM_SUBCORES
_CHUNK = 120


_LN2 = 0.6931471805599453
_SQRT2 = 1.4142135623730951

_ROT = ((13, 15, 26, 6), (17, 29, 16, 24))


def _ln(x):
    bits = lax.bitcast_convert_type(x, jnp.int32)
    e = ((bits >> 23) & 0xFF) - 127
    mant = lax.bitcast_convert_type((bits & 0x007FFFFF) | 0x3F800000,
                                    jnp.float32)
    big = mant > _SQRT2
    ione = jnp.ones((_LANES,), jnp.int32)
    izero = jnp.zeros((_LANES,), jnp.int32)
    mant = jnp.where(big, mant * 0.5, mant)
    e = (e + jnp.where(big, ione, izero)).astype(jnp.float32)
    s = (mant - 1.0) / (mant + 1.0)
    s2 = s * s
    p = 2.0 + s2 * (2.0 / 3.0 + s2 * (0.4 + s2 * (2.0 / 7.0)))
    return e * _LN2 + s * p


def _tf2x32(k1, k2, x1, x2):
    u32 = np.uint32
    ks = (u32(k1), u32(k2), u32(u32(k1) ^ u32(k2) ^ u32(0x1BD11BDA)))
    x0 = (x1 + ks[0]).astype(np.uint32)
    x1 = (x2 + ks[1]).astype(np.uint32)
    sched = ((0, 1, 2, 1), (1, 2, 0, 2), (0, 0, 1, 3), (1, 1, 2, 4),
             (0, 2, 0, 5))
    for g, a, b, inc in sched:
        for r in _ROT[g]:
            x0 = (x0 + x1).astype(np.uint32)
            x1 = x0 ^ ((x1 << u32(r)) | (x1 >> u32(32 - r)))
        x0 = (x0 + ks[a]).astype(np.uint32)
        x1 = (x1 + ks[b] + u32(inc)).astype(np.uint32)
    return x0, x1


@functools.lru_cache(maxsize=None)
def _negs_const(B, C, m):
    n = B * _NUM_NEG
    b1, b2 = _tf2x32(0, 42, np.zeros(2, np.uint32),
                     np.arange(2, dtype=np.uint32))
    cnt = np.arange(n, dtype=np.uint64)
    chi = (cnt >> np.uint64(32)).astype(np.uint32)
    clo = (cnt & np.uint64(0xFFFFFFFF)).astype(np.uint32)
    h1, h2 = _tf2x32(b1[0], b2[0], chi, clo)
    l1, l2 = _tf2x32(b1[1], b2[1], chi, clo)
    higher, lower = h1 ^ h2, l1 ^ l2
    span = np.uint32(C - m)
    mult = int(np.uint32(65536) % span)
    mult = np.uint32((mult * mult) % (1 << 32) % int(span))
    off = ((higher % span) * mult + (lower % span)).astype(np.uint32) % span
    negs = off.astype(np.int32).reshape(B, _NUM_NEG)
    t = (negs.reshape(B // _LANES, _LANES, _NUM_NEG).transpose(0, 2, 1)
         .reshape(B // _LANES, _NUM_NEG * _LANES))
    gp_w = B // _LANES // _NW
    out = np.zeros((_NW, gp_w, 512), np.int32)
    out[:, :, :_NUM_NEG * _LANES] = t.reshape(_NW, gp_w, _NUM_NEG * _LANES)
    return np.ascontiguousarray(out.reshape(_NW, 8, 128))


def _sc_stats_call(B, C, m):
    gp_w = B // _LANES // _NW
    per_w = gp_w * _NCAND * _LANES
    nper_w = gp_w * _NUM_NEG * _LANES
    n_chunks = per_w // _CHUNK
    assert per_w % _CHUNK == 0
    trow = (B // 128) * 1024
    mesh = plsc.VectorSubcoreMesh(
        core_axis_name="c", subcore_axis_name="s",
        num_cores=_NUM_CORES, num_subcores=_NUM_SUBCORES)

    g_chunks = (_NCAND * _LANES) // _CHUNK
    assert (_NCAND * _LANES) % _CHUNK == 0

    def body(scores_hbm, labt_hbm, negs_hbm, out_hbm,
             lab_v, negs_v, sort_v, cand_v, idx_v, vals_v, stat_v,
             sem, gsem):
        wid = lax.axis_index("s") * _NUM_CORES + lax.axis_index("c")
        span = gp_w * _LANES
        blk = wid >> 2
        sub = (wid & 3) * span
        cl = pltpu.async_copy(labt_hbm.at[:, pl.ds(blk * 128, 128)],
                              lab_v, sem)
        cn = pltpu.async_copy(negs_hbm.at[wid], negs_v, sem)
        cl.wait()
        cn.wait()
        gcopies = [[] for _ in range(gp_w)]
        for gl in range(gp_w):
            goff = gl * _LANES
            b = (wid * span + goff + lax.iota(jnp.int32, _LANES))
            b_term = ((b >> 7) << 10) + (b & 127)
            def _stage_step(t, _):
                sort_v[pl.ds(t * _LANES, _LANES)] = (
                    lab_v[t, pl.ds(sub + goff, _LANES)])
                return 0

            lax.fori_loop(0, m, _stage_step, 0)
            x0c = lab_v[m - 1, pl.ds(sub + goff, _LANES)]

            def _sort_round(rnd, _):
                par = rnd & 1

                def _exch(k, __):
                    p = jnp.minimum(par + 2 * k, m - 2)
                    a = sort_v[pl.ds(p * _LANES, _LANES)]
                    b = sort_v[pl.ds((p + 1) * _LANES, _LANES)]
                    sort_v[pl.ds(p * _LANES, _LANES)] = jnp.minimum(a, b)
                    sort_v[pl.ds((p + 1) * _LANES, _LANES)] = jnp.maximum(a, b)
                    return 0

                return lax.fori_loop(0, m // 2, _exch, 0)

            lax.fori_loop(0, m, _sort_round, 0)

            one = jnp.ones((_LANES,), jnp.int32)
            zero = jnp.zeros((_LANES,), jnp.int32)
            nvs = []
            for j in range(_NUM_NEG):
                f = gl * 512 + j * _LANES
                nvs.append(negs_v[f // 128, pl.ds(f % 128, _LANES)])

            def _shift_step(t, carry):
                e = sort_v[pl.ds(t * _LANES, _LANES)]
                return tuple(nv + jnp.where(nv >= e, one, zero)
                             for nv in carry)

            nvs = list(lax.fori_loop(0, m, _shift_step, tuple(nvs)))
            cands = [x0c] + nvs
            ibase = gl * (_NCAND * _LANES)
            for j in range(_NCAND):
                cand_v[pl.ds(j * _LANES, _LANES)] = cands[j]

            def _off_step(j, _):
                c = cand_v[pl.ds(j * _LANES, _LANES)]
                idx_v[pl.ds(ibase + j * _LANES, _LANES)] = (
                    ((c >> 3) * trow) + ((c & 7) << 7) + b_term)
                return 0

            lax.fori_loop(0, _NCAND, _off_step, 0)
            gs = gsem if gl == 0 else sem
            for ch in range(g_chunks):
                off = ibase + ch * _CHUNK
                gcopies[gl].append(pltpu.async_copy(
                    scores_hbm.at[idx_v.at[pl.ds(off, _CHUNK)]],
                    vals_v.at[pl.ds(off, _CHUNK)], gs))
        parts = [jnp.zeros((_LANES,), jnp.float32) for _ in range(8)]
        for gl in range(gp_w):
            for cc in gcopies[gl]:
                cc.wait()
            goff = gl * (_NCAND * _LANES)
            x0 = vals_v[pl.ds(goff, _LANES)]
            fone = jnp.ones((_LANES,), jnp.float32)
            fzero = jnp.zeros((_LANES,), jnp.float32)

            def _rk_step(j, carry):
                rank, mx = carry
                v = vals_v[pl.ds(goff + j * _LANES, _LANES)]
                return (rank + jnp.where(v > x0, fone, fzero),
                        jnp.maximum(mx, v))

            rank, mx = lax.fori_loop(1, _NCAND, _rk_step, (fzero, x0))

            def _se_step(j, carry):
                v = vals_v[pl.ds(goff + j * _LANES, _LANES)]
                return carry + jnp.exp(v - mx)

            se = lax.fori_loop(0, _NCAND, _se_step, fzero)
            w = _LN2 / _ln(rank + 2.0)
            loss_t = _ln(se) + (mx - x0)
            mrr_t = fone / (rank + 1.0)
            for i, k in enumerate(_KS):
                hit = jnp.where(rank < k, fone, fzero)
                parts[2 * i] = parts[2 * i] + hit
                parts[2 * i + 1] = parts[2 * i + 1] + hit * w
            parts[6] = parts[6] + mrr_t
            parts[7] = parts[7] + loss_t
        for k in range(8):
            stat_v[pl.ds(k * _LANES, _LANES)] = parts[k]
        pltpu.sync_copy(stat_v, out_hbm.at[wid])

    return pl.kernel(
        body,
        out_type=jax.ShapeDtypeStruct((_NW, 8 * _LANES), jnp.float32),
        mesh=mesh,
        scratch_types=[
            pltpu.VMEM((m, 128), jnp.int32),
            pltpu.VMEM((8, 128), jnp.int32),
            pltpu.VMEM((m * _LANES,), jnp.int32),
            pltpu.VMEM((_NCAND * _LANES,), jnp.int32),
            pltpu.VMEM((per_w,), jnp.int32),
            pltpu.VMEM((per_w,), jnp.float32),
            pltpu.VMEM((8 * _LANES,), jnp.float32),
            pltpu.SemaphoreType.DMA,
            pltpu.SemaphoreType.DMA,
        ],
    )


def _tc_reduce_call(parts, B):
    inv = 1.0 / B

    def body(st_ref, out_ref):
        tot = jnp.sum(st_ref[...], axis=0, keepdims=True)
        for k in range(8):
            out_ref[k] = jnp.sum(tot[:, k * _LANES:(k + 1) * _LANES]) * inv

    return pl.pallas_call(
        body,
        out_shape=jax.ShapeDtypeStruct((8,), jnp.float32),
        out_specs=pl.BlockSpec(memory_space=pltpu.SMEM),
    )(parts)


def kernel(scores, labels):
    B, C = scores.shape
    m = labels.shape[1]
    negs = jnp.asarray(_negs_const(B, C, m))
    labt = labels.T
    phys = (scores.reshape(B // 128, 128, C // 8, 8)
            .transpose(2, 0, 3, 1).reshape(-1))
    parts = _sc_stats_call(B, C, m)(phys, labt, negs)
    return _tc_reduce_call(parts, B)


# speedup vs baseline: 1.0109x; 1.0109x over previous
import functools
import math

import jax
import jax.numpy as jnp
import numpy as np
from jax import lax
from jax.experimental import pallas as pl
from jax.experimental.pallas import tpu as pltpu
from jax.experimental.pallas import tpu_sc as plsc

_NUM_NEG = 29
_NCAND = _NUM_NEG + 1
_KS = (1, 5, 10)
_LANES = 16
_NUM_CORES = 2
_NUM_SUBCORES = 16
_NW = _NUM_CORES * _NUM_SUBCORES
_CHUNK = 120


_LN2 = 0.6931471805599453
_SQRT2 = 1.4142135623730951

_ROT = ((13, 15, 26, 6), (17, 29, 16, 24))


def _ln(x):
    bits = lax.bitcast_convert_type(x, jnp.int32)
    e = ((bits >> 23) & 0xFF) - 127
    mant = lax.bitcast_convert_type((bits & 0x007FFFFF) | 0x3F800000,
                                    jnp.float32)
    big = mant > _SQRT2
    ione = jnp.ones((_LANES,), jnp.int32)
    izero = jnp.zeros((_LANES,), jnp.int32)
    mant = jnp.where(big, mant * 0.5, mant)
    e = (e + jnp.where(big, ione, izero)).astype(jnp.float32)
    s = (mant - 1.0) / (mant + 1.0)
    s2 = s * s
    p = 2.0 + s2 * (2.0 / 3.0 + s2 * (0.4 + s2 * (2.0 / 7.0)))
    return e * _LN2 + s * p


def _tf2x32(k1, k2, x1, x2):
    u32 = np.uint32
    ks = (u32(k1), u32(k2), u32(u32(k1) ^ u32(k2) ^ u32(0x1BD11BDA)))
    x0 = (x1 + ks[0]).astype(np.uint32)
    x1 = (x2 + ks[1]).astype(np.uint32)
    sched = ((0, 1, 2, 1), (1, 2, 0, 2), (0, 0, 1, 3), (1, 1, 2, 4),
             (0, 2, 0, 5))
    for g, a, b, inc in sched:
        for r in _ROT[g]:
            x0 = (x0 + x1).astype(np.uint32)
            x1 = x0 ^ ((x1 << u32(r)) | (x1 >> u32(32 - r)))
        x0 = (x0 + ks[a]).astype(np.uint32)
        x1 = (x1 + ks[b] + u32(inc)).astype(np.uint32)
    return x0, x1


@functools.lru_cache(maxsize=None)
def _negs_const(B, C, m):
    n = B * _NUM_NEG
    b1, b2 = _tf2x32(0, 42, np.zeros(2, np.uint32),
                     np.arange(2, dtype=np.uint32))
    cnt = np.arange(n, dtype=np.uint64)
    chi = (cnt >> np.uint64(32)).astype(np.uint32)
    clo = (cnt & np.uint64(0xFFFFFFFF)).astype(np.uint32)
    h1, h2 = _tf2x32(b1[0], b2[0], chi, clo)
    l1, l2 = _tf2x32(b1[1], b2[1], chi, clo)
    higher, lower = h1 ^ h2, l1 ^ l2
    span = np.uint32(C - m)
    mult = int(np.uint32(65536) % span)
    mult = np.uint32((mult * mult) % (1 << 32) % int(span))
    off = ((higher % span) * mult + (lower % span)).astype(np.uint32) % span
    negs = off.astype(np.int32).reshape(B, _NUM_NEG)
    t = (negs.reshape(B // _LANES, _LANES, _NUM_NEG).transpose(0, 2, 1)
         .reshape(B // _LANES, _NUM_NEG * _LANES))
    gp_w = B // _LANES // _NW
    out = np.zeros((_NW, gp_w, 512), np.int32)
    out[:, :, :_NUM_NEG * _LANES] = t.reshape(_NW, gp_w, _NUM_NEG * _LANES)
    return np.ascontiguousarray(out.reshape(_NW, 8, 128))


def _sc_stats_call(B, C, m):
    gp_w = B // _LANES // _NW
    per_w = gp_w * _NCAND * _LANES
    nper_w = gp_w * _NUM_NEG * _LANES
    n_chunks = per_w // _CHUNK
    assert per_w % _CHUNK == 0
    trow = (B // 128) * 1024
    mesh = plsc.VectorSubcoreMesh(
        core_axis_name="c", subcore_axis_name="s",
        num_cores=_NUM_CORES, num_subcores=_NUM_SUBCORES)

    g_chunks = (_NCAND * _LANES) // _CHUNK
    assert (_NCAND * _LANES) % _CHUNK == 0

    def body(scores_hbm, labt_hbm, negs_hbm, out_hbm,
             lab_v, negs_v, sort_v, cand_v, idx_v, vals_v, stat_v,
             sem, gsem):
        wid = lax.axis_index("s") * _NUM_CORES + lax.axis_index("c")
        span = gp_w * _LANES
        blk = wid >> 2
        sub = (wid & 3) * span
        cl = pltpu.async_copy(labt_hbm.at[:, pl.ds(blk * 128, 128)],
                              lab_v, sem)
        cn = pltpu.async_copy(negs_hbm.at[wid], negs_v, sem)
        cl.wait()
        cn.wait()
        gcopies = [[] for _ in range(gp_w)]
        for gl in range(gp_w):
            goff = gl * _LANES
            b = (wid * span + goff + lax.iota(jnp.int32, _LANES))
            b_term = ((b >> 7) << 10) + (b & 127)
            svs = [lab_v[t, pl.ds(sub + goff, _LANES)] for t in range(m)]
            x0c = svs[m - 1]
            for rnd in range(m):
                for t in range(rnd & 1, m - 1, 2):
                    lo = jnp.minimum(svs[t], svs[t + 1])
                    hi = jnp.maximum(svs[t], svs[t + 1])
                    svs[t], svs[t + 1] = lo, hi
            for t in range(m):
                sort_v[pl.ds(t * _LANES, _LANES)] = svs[t]
            one = jnp.ones((_LANES,), jnp.int32)
            zero = jnp.zeros((_LANES,), jnp.int32)
            nvs = []
            for j in range(_NUM_NEG):
                f = gl * 512 + j * _LANES
                nvs.append(negs_v[f // 128, pl.ds(f % 128, _LANES)])

            def _shift_step(t, carry):
                e = sort_v[pl.ds(t * _LANES, _LANES)]
                return tuple(nv + jnp.where(nv >= e, one, zero)
                             for nv in carry)

            nvs = list(lax.fori_loop(0, m, _shift_step, tuple(nvs)))
            cands = [x0c] + nvs
            ibase = gl * (_NCAND * _LANES)
            for j in range(_NCAND):
                cand_v[pl.ds(j * _LANES, _LANES)] = cands[j]

            def _off_step(j, _):
                c = cand_v[pl.ds(j * _LANES, _LANES)]
                idx_v[pl.ds(ibase + j * _LANES, _LANES)] = (
                    ((c >> 3) * trow) + ((c & 7) << 7) + b_term)
                return 0

            lax.fori_loop(0, _NCAND, _off_step, 0)
            gs = gsem if gl == 0 else sem
            for ch in range(g_chunks):
                off = ibase + ch * _CHUNK
                gcopies[gl].append(pltpu.async_copy(
                    scores_hbm.at[idx_v.at[pl.ds(off, _CHUNK)]],
                    vals_v.at[pl.ds(off, _CHUNK)], gs))
        parts = [jnp.zeros((_LANES,), jnp.float32) for _ in range(8)]
        for gl in range(gp_w):
            for cc in gcopies[gl]:
                cc.wait()
            goff = gl * (_NCAND * _LANES)
            x0 = vals_v[pl.ds(goff, _LANES)]
            fone = jnp.ones((_LANES,), jnp.float32)
            fzero = jnp.zeros((_LANES,), jnp.float32)

            def _rk_step(j, carry):
                rank, mx = carry
                v = vals_v[pl.ds(goff + j * _LANES, _LANES)]
                return (rank + jnp.where(v > x0, fone, fzero),
                        jnp.maximum(mx, v))

            rank, mx = lax.fori_loop(1, _NCAND, _rk_step, (fzero, x0))

            def _se_step(j, carry):
                v = vals_v[pl.ds(goff + j * _LANES, _LANES)]
                return carry + jnp.exp(v - mx)

            se = lax.fori_loop(0, _NCAND, _se_step, fzero)
            w = _LN2 / _ln(rank + 2.0)
            loss_t = _ln(se) + (mx - x0)
            mrr_t = fone / (rank + 1.0)
            for i, k in enumerate(_KS):
                hit = jnp.where(rank < k, fone, fzero)
                parts[2 * i] = parts[2 * i] + hit
                parts[2 * i + 1] = parts[2 * i + 1] + hit * w
            parts[6] = parts[6] + mrr_t
            parts[7] = parts[7] + loss_t
        for k in range(8):
            stat_v[pl.ds(k * _LANES, _LANES)] = parts[k]
        pltpu.sync_copy(stat_v, out_hbm.at[wid])

    return pl.kernel(
        body,
        out_type=jax.ShapeDtypeStruct((_NW, 8 * _LANES), jnp.float32),
        mesh=mesh,
        scratch_types=[
            pltpu.VMEM((m, 128), jnp.int32),
            pltpu.VMEM((8, 128), jnp.int32),
            pltpu.VMEM((m * _LANES,), jnp.int32),
            pltpu.VMEM((_NCAND * _LANES,), jnp.int32),
            pltpu.VMEM((per_w,), jnp.int32),
            pltpu.VMEM((per_w,), jnp.float32),
            pltpu.VMEM((8 * _LANES,), jnp.float32),
            pltpu.SemaphoreType.DMA,
            pltpu.SemaphoreType.DMA,
        ],
    )


def _tc_reduce_call(parts, B):
    inv = 1.0 / B

    def body(st_ref, out_ref):
        tot = jnp.sum(st_ref[...], axis=0, keepdims=True)
        for k in range(8):
            out_ref[k] = jnp.sum(tot[:, k * _LANES:(k + 1) * _LANES]) * inv

    return pl.pallas_call(
        body,
        out_shape=jax.ShapeDtypeStruct((8,), jnp.float32),
        out_specs=pl.BlockSpec(memory_space=pltpu.SMEM),
    )(parts)


def kernel(scores, labels):
    B, C = scores.shape
    m = labels.shape[1]
    negs = jnp.asarray(_negs_const(B, C, m))
    labt = labels.T
    phys = (scores.reshape(B // 128, 128, C // 8, 8)
            .transpose(2, 0, 3, 1).reshape(-1))
    parts = _sc_stats_call(B, C, m)(phys, labt, negs)
    return _tc_reduce_call(parts, B)


# speedup vs baseline: 1.0290x; 1.0179x over previous
import functools
import math

import jax
import jax.numpy as jnp
import numpy as np
from jax import lax
from jax.experimental import pallas as pl
from jax.experimental.pallas import tpu as pltpu
from jax.experimental.pallas import tpu_sc as plsc

_NUM_NEG = 29
_NCAND = _NUM_NEG + 1
_KS = (1, 5, 10)
_LANES = 16
_NUM_CORES = 2
_NUM_SUBCORES = 16
_NW = _NUM_CORES * _NUM_SUBCORES
_CHUNK = 120


_LN2 = 0.6931471805599453
_SQRT2 = 1.4142135623730951

_ROT = ((13, 15, 26, 6), (17, 29, 16, 24))


@functools.lru_cache(maxsize=None)
def _batcher_pairs(n):
    pairs = []
    p = 1
    while p < n:
        k = p
        while k >= 1:
            for j in range(k % p, n - k, 2 * k):
                for i in range(min(k, n - j - k)):
                    if (i + j) // (2 * p) == (i + j + k) // (2 * p):
                        pairs.append((i + j, i + j + k))
            k //= 2
        p *= 2
    return tuple(pairs)


def _ln(x):
    bits = lax.bitcast_convert_type(x, jnp.int32)
    e = ((bits >> 23) & 0xFF) - 127
    mant = lax.bitcast_convert_type((bits & 0x007FFFFF) | 0x3F800000,
                                    jnp.float32)
    big = mant > _SQRT2
    ione = jnp.ones((_LANES,), jnp.int32)
    izero = jnp.zeros((_LANES,), jnp.int32)
    mant = jnp.where(big, mant * 0.5, mant)
    e = (e + jnp.where(big, ione, izero)).astype(jnp.float32)
    s = (mant - 1.0) / (mant + 1.0)
    s2 = s * s
    p = 2.0 + s2 * (2.0 / 3.0 + s2 * (0.4 + s2 * (2.0 / 7.0)))
    return e * _LN2 + s * p


def _tf2x32(k1, k2, x1, x2):
    u32 = np.uint32
    ks = (u32(k1), u32(k2), u32(u32(k1) ^ u32(k2) ^ u32(0x1BD11BDA)))
    x0 = (x1 + ks[0]).astype(np.uint32)
    x1 = (x2 + ks[1]).astype(np.uint32)
    sched = ((0, 1, 2, 1), (1, 2, 0, 2), (0, 0, 1, 3), (1, 1, 2, 4),
             (0, 2, 0, 5))
    for g, a, b, inc in sched:
        for r in _ROT[g]:
            x0 = (x0 + x1).astype(np.uint32)
            x1 = x0 ^ ((x1 << u32(r)) | (x1 >> u32(32 - r)))
        x0 = (x0 + ks[a]).astype(np.uint32)
        x1 = (x1 + ks[b] + u32(inc)).astype(np.uint32)
    return x0, x1


@functools.lru_cache(maxsize=None)
def _negs_const(B, C, m):
    n = B * _NUM_NEG
    b1, b2 = _tf2x32(0, 42, np.zeros(2, np.uint32),
                     np.arange(2, dtype=np.uint32))
    cnt = np.arange(n, dtype=np.uint64)
    chi = (cnt >> np.uint64(32)).astype(np.uint32)
    clo = (cnt & np.uint64(0xFFFFFFFF)).astype(np.uint32)
    h1, h2 = _tf2x32(b1[0], b2[0], chi, clo)
    l1, l2 = _tf2x32(b1[1], b2[1], chi, clo)
    higher, lower = h1 ^ h2, l1 ^ l2
    span = np.uint32(C - m)
    mult = int(np.uint32(65536) % span)
    mult = np.uint32((mult * mult) % (1 << 32) % int(span))
    off = ((higher % span) * mult + (lower % span)).astype(np.uint32) % span
    negs = off.astype(np.int32).reshape(B, _NUM_NEG)
    t = (negs.reshape(B // _LANES, _LANES, _NUM_NEG).transpose(0, 2, 1)
         .reshape(B // _LANES, _NUM_NEG * _LANES))
    gp_w = B // _LANES // _NW
    out = np.zeros((_NW, gp_w, 512), np.int32)
    out[:, :, :_NUM_NEG * _LANES] = t.reshape(_NW, gp_w, _NUM_NEG * _LANES)
    return np.ascontiguousarray(out.reshape(_NW, 8, 128))


def _sc_stats_call(B, C, m):
    gp_w = B // _LANES // _NW
    per_w = gp_w * _NCAND * _LANES
    nper_w = gp_w * _NUM_NEG * _LANES
    n_chunks = per_w // _CHUNK
    assert per_w % _CHUNK == 0
    trow = (B // 128) * 1024
    mesh = plsc.VectorSubcoreMesh(
        core_axis_name="c", subcore_axis_name="s",
        num_cores=_NUM_CORES, num_subcores=_NUM_SUBCORES)

    g_chunks = (_NCAND * _LANES) // _CHUNK
    assert (_NCAND * _LANES) % _CHUNK == 0

    def body(scores_hbm, labt_hbm, negs_hbm, out_hbm,
             lab_v, negs_v, sort_v, cand_v, idx_v, vals_v, stat_v,
             sem, gsem):
        wid = lax.axis_index("s") * _NUM_CORES + lax.axis_index("c")
        span = gp_w * _LANES
        blk = wid >> 2
        sub = (wid & 3) * span
        cl = pltpu.async_copy(labt_hbm.at[:, pl.ds(blk * 128, 128)],
                              lab_v, sem)
        cn = pltpu.async_copy(negs_hbm.at[wid], negs_v, sem)
        cl.wait()
        cn.wait()
        gcopies = [[] for _ in range(gp_w)]
        for gl in range(gp_w):
            goff = gl * _LANES
            b = (wid * span + goff + lax.iota(jnp.int32, _LANES))
            b_term = ((b >> 7) << 10) + (b & 127)
            svs = [lab_v[t, pl.ds(sub + goff, _LANES)] for t in range(m)]
            x0c = svs[m - 1]
            for t, t2 in _batcher_pairs(m):
                lo = jnp.minimum(svs[t], svs[t2])
                hi = jnp.maximum(svs[t], svs[t2])
                svs[t], svs[t2] = lo, hi
            for t in range(m):
                sort_v[pl.ds(t * _LANES, _LANES)] = svs[t]
            one = jnp.ones((_LANES,), jnp.int32)
            zero = jnp.zeros((_LANES,), jnp.int32)
            nvs = []
            for j in range(_NUM_NEG):
                f = gl * 512 + j * _LANES
                nvs.append(negs_v[f // 128, pl.ds(f % 128, _LANES)])

            def _shift_step(t, carry):
                e = sort_v[pl.ds(t * _LANES, _LANES)]
                return tuple(nv + jnp.where(nv >= e, one, zero)
                             for nv in carry)

            nvs = list(lax.fori_loop(0, m, _shift_step, tuple(nvs)))
            cands = [x0c] + nvs
            ibase = gl * (_NCAND * _LANES)
            for j in range(_NCAND):
                cand_v[pl.ds(j * _LANES, _LANES)] = cands[j]

            def _off_step(j, _):
                c = cand_v[pl.ds(j * _LANES, _LANES)]
                idx_v[pl.ds(ibase + j * _LANES, _LANES)] = (
                    ((c >> 3) * trow) + ((c & 7) << 7) + b_term)
                return 0

            lax.fori_loop(0, _NCAND, _off_step, 0)
            gs = gsem if gl == 0 else sem
            for ch in range(g_chunks):
                off = ibase + ch * _CHUNK
                gcopies[gl].append(pltpu.async_copy(
                    scores_hbm.at[idx_v.at[pl.ds(off, _CHUNK)]],
                    vals_v.at[pl.ds(off, _CHUNK)], gs))
        parts = [jnp.zeros((_LANES,), jnp.float32) for _ in range(8)]
        for gl in range(gp_w):
            for cc in gcopies[gl]:
                cc.wait()
            goff = gl * (_NCAND * _LANES)
            x0 = vals_v[pl.ds(goff, _LANES)]
            fone = jnp.ones((_LANES,), jnp.float32)
            fzero = jnp.zeros((_LANES,), jnp.float32)

            def _rk_step(j, carry):
                rank, mx = carry
                v = vals_v[pl.ds(goff + j * _LANES, _LANES)]
                return (rank + jnp.where(v > x0, fone, fzero),
                        jnp.maximum(mx, v))

            rank, mx = lax.fori_loop(1, _NCAND, _rk_step, (fzero, x0))

            def _se_step(j, carry):
                v = vals_v[pl.ds(goff + j * _LANES, _LANES)]
                return carry + jnp.exp(v - mx)

            se = lax.fori_loop(0, _NCAND, _se_step, fzero)
            w = _LN2 / _ln(rank + 2.0)
            loss_t = _ln(se) + (mx - x0)
            mrr_t = fone / (rank + 1.0)
            for i, k in enumerate(_KS):
                hit = jnp.where(rank < k, fone, fzero)
                parts[2 * i] = parts[2 * i] + hit
                parts[2 * i + 1] = parts[2 * i + 1] + hit * w
            parts[6] = parts[6] + mrr_t
            parts[7] = parts[7] + loss_t
        for k in range(8):
            stat_v[pl.ds(k * _LANES, _LANES)] = parts[k]
        pltpu.sync_copy(stat_v, out_hbm.at[wid])

    return pl.kernel(
        body,
        out_type=jax.ShapeDtypeStruct((_NW, 8 * _LANES), jnp.float32),
        mesh=mesh,
        scratch_types=[
            pltpu.VMEM((m, 128), jnp.int32),
            pltpu.VMEM((8, 128), jnp.int32),
            pltpu.VMEM((m * _LANES,), jnp.int32),
            pltpu.VMEM((_NCAND * _LANES,), jnp.int32),
            pltpu.VMEM((per_w,), jnp.int32),
            pltpu.VMEM((per_w,), jnp.float32),
            pltpu.VMEM((8 * _LANES,), jnp.float32),
            pltpu.SemaphoreType.DMA,
            pltpu.SemaphoreType.DMA,
        ],
    )


def _tc_reduce_call(parts, B):
    inv = 1.0 / B

    def body(st_ref, out_ref):
        tot = jnp.sum(st_ref[...], axis=0, keepdims=True)
        for k in range(8):
            out_ref[k] = jnp.sum(tot[:, k * _LANES:(k + 1) * _LANES]) * inv

    return pl.pallas_call(
        body,
        out_shape=jax.ShapeDtypeStruct((8,), jnp.float32),
        out_specs=pl.BlockSpec(memory_space=pltpu.SMEM),
    )(parts)


def kernel(scores, labels):
    B, C = scores.shape
    m = labels.shape[1]
    negs = jnp.asarray(_negs_const(B, C, m))
    labt = labels.T
    phys = (scores.reshape(B // 128, 128, C // 8, 8)
            .transpose(2, 0, 3, 1).reshape(-1))
    parts = _sc_stats_call(B, C, m)(phys, labt, negs)
    return _tc_reduce_call(parts, B)


# speedup vs baseline: 1.0308x; 1.0017x over previous
import functools

import jax
import jax.numpy as jnp
import numpy as np
from jax import lax
from jax.experimental import pallas as pl
from jax.experimental.pallas import tpu as pltpu
from jax.experimental.pallas import tpu_sc as plsc

_NUM_NEG = 29
_NCAND = _NUM_NEG + 1
_KS = (1, 5, 10)
_LANES = 16
_NUM_CORES = 2
_NUM_SUBCORES = 16
_NW = _NUM_CORES * _NUM_SUBCORES
_CHUNK = 120


_LN2 = 0.6931471805599453
_SQRT2 = 1.4142135623730951

_ROT = ((13, 15, 26, 6), (17, 29, 16, 24))


@functools.lru_cache(maxsize=None)
def _batcher_pairs(n):
    pairs = []
    p = 1
    while p < n:
        k = p
        while k >= 1:
            for j in range(k % p, n - k, 2 * k):
                for i in range(min(k, n - j - k)):
                    if (i + j) // (2 * p) == (i + j + k) // (2 * p):
                        pairs.append((i + j, i + j + k))
            k //= 2
        p *= 2
    return tuple(pairs)


def _ln(x):
    bits = lax.bitcast_convert_type(x, jnp.int32)
    e = ((bits >> 23) & 0xFF) - 127
    mant = lax.bitcast_convert_type((bits & 0x007FFFFF) | 0x3F800000,
                                    jnp.float32)
    big = mant > _SQRT2
    ione = jnp.ones((_LANES,), jnp.int32)
    izero = jnp.zeros((_LANES,), jnp.int32)
    mant = jnp.where(big, mant * 0.5, mant)
    e = (e + jnp.where(big, ione, izero)).astype(jnp.float32)
    s = (mant - 1.0) / (mant + 1.0)
    s2 = s * s
    p = 2.0 + s2 * (2.0 / 3.0 + s2 * (0.4 + s2 * (2.0 / 7.0)))
    return e * _LN2 + s * p


def _tf2x32(k1, k2, x1, x2):
    u32 = np.uint32
    ks = (u32(k1), u32(k2), u32(u32(k1) ^ u32(k2) ^ u32(0x1BD11BDA)))
    x0 = (x1 + ks[0]).astype(np.uint32)
    x1 = (x2 + ks[1]).astype(np.uint32)
    sched = ((0, 1, 2, 1), (1, 2, 0, 2), (0, 0, 1, 3), (1, 1, 2, 4),
             (0, 2, 0, 5))
    for g, a, b, inc in sched:
        for r in _ROT[g]:
            x0 = (x0 + x1).astype(np.uint32)
            x1 = x0 ^ ((x1 << u32(r)) | (x1 >> u32(32 - r)))
        x0 = (x0 + ks[a]).astype(np.uint32)
        x1 = (x1 + ks[b] + u32(inc)).astype(np.uint32)
    return x0, x1


@functools.lru_cache(maxsize=None)
def _negs_const(B, C, m):
    n = B * _NUM_NEG
    b1, b2 = _tf2x32(0, 42, np.zeros(2, np.uint32),
                     np.arange(2, dtype=np.uint32))
    cnt = np.arange(n, dtype=np.uint64)
    chi = (cnt >> np.uint64(32)).astype(np.uint32)
    clo = (cnt & np.uint64(0xFFFFFFFF)).astype(np.uint32)
    h1, h2 = _tf2x32(b1[0], b2[0], chi, clo)
    l1, l2 = _tf2x32(b1[1], b2[1], chi, clo)
    higher, lower = h1 ^ h2, l1 ^ l2
    span = np.uint32(C - m)
    mult = int(np.uint32(65536) % span)
    mult = np.uint32((mult * mult) % (1 << 32) % int(span))
    off = ((higher % span) * mult + (lower % span)).astype(np.uint32) % span
    negs = off.astype(np.int32).reshape(B, _NUM_NEG)
    t = (negs.reshape(B // _LANES, _LANES, _NUM_NEG).transpose(0, 2, 1)
         .reshape(B // _LANES, _NUM_NEG * _LANES))
    gp_w = B // _LANES // _NW
    out = np.zeros((_NW, gp_w, 512), np.int32)
    out[:, :, :_NUM_NEG * _LANES] = t.reshape(_NW, gp_w, _NUM_NEG * _LANES)
    return np.ascontiguousarray(out.reshape(_NW, 8, 128))


def _sc_stats_call(B, C, m):
    gp_w = B // _LANES // _NW
    per_w = gp_w * _NCAND * _LANES
    trow = (B // 128) * 1024
    mesh = plsc.VectorSubcoreMesh(
        core_axis_name="c", subcore_axis_name="s",
        num_cores=_NUM_CORES, num_subcores=_NUM_SUBCORES)

    g_chunks = (_NCAND * _LANES) // _CHUNK
    assert (_NCAND * _LANES) % _CHUNK == 0

    def body(scores_hbm, labt_hbm, negs_hbm, out_hbm,
             lab_v, negs_v, sort_v, cand_v, idx_v, vals_v, stat_v,
             sem, gsem):
        wid = lax.axis_index("s") * _NUM_CORES + lax.axis_index("c")
        span = gp_w * _LANES
        blk = wid >> 2
        sub = (wid & 3) * span
        cl = pltpu.async_copy(labt_hbm.at[:, pl.ds(blk * 128, 128)],
                              lab_v, sem)
        cn = pltpu.async_copy(negs_hbm.at[wid], negs_v, sem)
        cl.wait()
        cn.wait()
        gcopies = [[] for _ in range(gp_w)]
        for gl in range(gp_w):
            goff = gl * _LANES
            b = (wid * span + goff + lax.iota(jnp.int32, _LANES))
            b_term = ((b >> 7) << 10) + (b & 127)
            svs = [lab_v[t, pl.ds(sub + goff, _LANES)] for t in range(m)]
            x0c = svs[m - 1]
            for t, t2 in _batcher_pairs(m):
                lo = jnp.minimum(svs[t], svs[t2])
                hi = jnp.maximum(svs[t], svs[t2])
                svs[t], svs[t2] = lo, hi
            for t in range(m):
                sort_v[pl.ds(t * _LANES, _LANES)] = svs[t]
            one = jnp.ones((_LANES,), jnp.int32)
            zero = jnp.zeros((_LANES,), jnp.int32)
            nvs = []
            for j in range(_NUM_NEG):
                f = gl * 512 + j * _LANES
                nvs.append(negs_v[f // 128, pl.ds(f % 128, _LANES)])

            def _shift_step(t, carry):
                e = sort_v[pl.ds(t * _LANES, _LANES)]
                return tuple(nv + jnp.where(nv >= e, one, zero)
                             for nv in carry)

            nvs = list(lax.fori_loop(0, m, _shift_step, tuple(nvs)))
            cands = [x0c] + nvs
            ibase = gl * (_NCAND * _LANES)
            for j in range(_NCAND):
                cand_v[pl.ds(j * _LANES, _LANES)] = cands[j]

            def _off_step(j, _):
                c = cand_v[pl.ds(j * _LANES, _LANES)]
                idx_v[pl.ds(ibase + j * _LANES, _LANES)] = (
                    ((c >> 3) * trow) + ((c & 7) << 7) + b_term)
                return 0

            lax.fori_loop(0, _NCAND, _off_step, 0)
            gs = gsem if gl == 0 else sem
            for ch in range(g_chunks):
                off = ibase + ch * _CHUNK
                gcopies[gl].append(pltpu.async_copy(
                    scores_hbm.at[idx_v.at[pl.ds(off, _CHUNK)]],
                    vals_v.at[pl.ds(off, _CHUNK)], gs))
        parts = [jnp.zeros((_LANES,), jnp.float32) for _ in range(8)]
        for gl in range(gp_w):
            for cc in gcopies[gl]:
                cc.wait()
            goff = gl * (_NCAND * _LANES)
            x0 = vals_v[pl.ds(goff, _LANES)]
            fone = jnp.ones((_LANES,), jnp.float32)
            fzero = jnp.zeros((_LANES,), jnp.float32)

            def _rk_step(j, carry):
                rank, mx = carry
                v = vals_v[pl.ds(goff + j * _LANES, _LANES)]
                return (rank + jnp.where(v > x0, fone, fzero),
                        jnp.maximum(mx, v))

            rank, mx = lax.fori_loop(1, _NCAND, _rk_step, (fzero, x0))

            def _se_step(j, carry):
                v = vals_v[pl.ds(goff + j * _LANES, _LANES)]
                return carry + jnp.exp(v - mx)

            se = lax.fori_loop(0, _NCAND, _se_step, fzero)
            w = _LN2 / _ln(rank + 2.0)
            loss_t = _ln(se) + (mx - x0)
            mrr_t = fone / (rank + 1.0)
            for i, k in enumerate(_KS):
                hit = jnp.where(rank < k, fone, fzero)
                parts[2 * i] = parts[2 * i] + hit
                parts[2 * i + 1] = parts[2 * i + 1] + hit * w
            parts[6] = parts[6] + mrr_t
            parts[7] = parts[7] + loss_t
        for k in range(8):
            stat_v[pl.ds(k * _LANES, _LANES)] = parts[k]
        pltpu.sync_copy(stat_v, out_hbm.at[wid])

    return pl.kernel(
        body,
        out_type=jax.ShapeDtypeStruct((_NW, 8 * _LANES), jnp.float32),
        mesh=mesh,
        scratch_types=[
            pltpu.VMEM((m, 128), jnp.int32),
            pltpu.VMEM((8, 128), jnp.int32),
            pltpu.VMEM((m * _LANES,), jnp.int32),
            pltpu.VMEM((_NCAND * _LANES,), jnp.int32),
            pltpu.VMEM((per_w,), jnp.int32),
            pltpu.VMEM((per_w,), jnp.float32),
            pltpu.VMEM((8 * _LANES,), jnp.float32),
            pltpu.SemaphoreType.DMA,
            pltpu.SemaphoreType.DMA,
        ],
    )


def _tc_reduce_call(parts, B):
    inv = 1.0 / B

    def body(st_ref, out_ref):
        tot = jnp.sum(st_ref[...], axis=0, keepdims=True)
        for k in range(8):
            out_ref[k] = jnp.sum(tot[:, k * _LANES:(k + 1) * _LANES]) * inv

    return pl.pallas_call(
        body,
        out_shape=jax.ShapeDtypeStruct((8,), jnp.float32),
        out_specs=pl.BlockSpec(memory_space=pltpu.SMEM),
    )(parts)


def kernel(scores, labels):
    B, C = scores.shape
    m = labels.shape[1]
    negs = jnp.asarray(_negs_const(B, C, m))
    labt = labels.T
    phys = (scores.reshape(B // 128, 128, C // 8, 8)
            .transpose(2, 0, 3, 1).reshape(-1))
    parts = _sc_stats_call(B, C, m)(phys, labt, negs)
    return _tc_reduce_call(parts, B)
